# Initial kernel scaffold; baseline (speedup 1.0000x reference)
import jax
import jax.numpy as jnp
from jax.experimental import pallas as pl


def kernel(X_num, X_cat, tables, W1, b1, W2, b2, W3, b3):
    raise NotImplementedError("write your pallas kernel here")


# speedup vs baseline: 7.3541x; 7.3541x over previous
import functools

import jax
import jax.numpy as jnp
from jax import lax
from jax.experimental import pallas as pl
from jax.experimental.pallas import tpu as pltpu
from jax.experimental.pallas import tpu_sc as plsc

_NUM_FIELDS = 26
_EMB = 32
_VOCAB = 100000
_NUM_NUM = 13


@functools.partial(jax.jit, static_argnums=(2, 3, 4))
def _sc_gather(flat_tables, flat_idx, n_total, emb, chunk):
    info = plsc.get_sparse_core_info()
    nw = info.num_cores * info.num_subcores
    per_w = n_total // nw
    n_chunks = per_w // chunk

    mesh = plsc.VectorSubcoreMesh(core_axis_name="c", subcore_axis_name="s")

    @functools.partial(
        pl.kernel,
        mesh=mesh,
        out_type=jax.ShapeDtypeStruct((n_total, emb), jnp.float32),
        scratch_types=[
            pltpu.VMEM((chunk,), jnp.int32),
            pltpu.VMEM((chunk, emb), jnp.float32),
            pltpu.SemaphoreType.DMA,
        ],
        compiler_params=pltpu.CompilerParams(use_tc_tiling_on_sc=False),
    )
    def gather_kernel(tab_hbm, idx_hbm, out_hbm, idx_v, rows_v, sem):
        wid = lax.axis_index("s") * info.num_cores + lax.axis_index("c")
        base = wid * per_w

        def body(i, carry):
            off = base + i * chunk
            pltpu.sync_copy(idx_hbm.at[pl.ds(off, chunk)], idx_v)
            pltpu.async_copy(tab_hbm.at[idx_v], rows_v, sem).wait()
            pltpu.sync_copy(rows_v, out_hbm.at[pl.ds(off, chunk)])
            return carry

        lax.fori_loop(0, n_chunks, body, 0)

    return gather_kernel(flat_tables, flat_idx)


def _mlp_body(xn_ref, xe_ref, w1n_ref, w1e_ref, b1_ref, w2_ref, b2_ref,
              w3_ref, b3_ref, out_ref):
    h = jnp.dot(xe_ref[...], w1e_ref[...], preferred_element_type=jnp.float32)
    h = h + jnp.dot(xn_ref[...], w1n_ref[...],
                    preferred_element_type=jnp.float32)
    h = jnp.maximum(h + b1_ref[...], 0.0)
    h = jnp.maximum(
        jnp.dot(h, w2_ref[...], preferred_element_type=jnp.float32)
        + b2_ref[...], 0.0)
    out_ref[...] = (jnp.dot(h, w3_ref[...], preferred_element_type=jnp.float32)
                    + b3_ref[...])


@functools.partial(jax.jit, static_argnums=(9,))
def _tc_mlp(x_num, x_emb, w1n, w1e, b1, w2, b2, w3, b3, blk):
    b = x_num.shape[0]
    nn = x_num.shape[1]
    de = x_emb.shape[1]
    grid = b // blk
    return pl.pallas_call(
        _mlp_body,
        grid=(grid,),
        in_specs=[
            pl.BlockSpec((blk, nn), lambda i: (i, 0)),
            pl.BlockSpec((blk, de), lambda i: (i, 0)),
            pl.BlockSpec((nn, 32), lambda i: (0, 0)),
            pl.BlockSpec((de, 32), lambda i: (0, 0)),
            pl.BlockSpec((1, 32), lambda i: (0, 0)),
            pl.BlockSpec((32, 16), lambda i: (0, 0)),
            pl.BlockSpec((1, 16), lambda i: (0, 0)),
            pl.BlockSpec((16, 1), lambda i: (0, 0)),
            pl.BlockSpec((1, 1), lambda i: (0, 0)),
        ],
        out_specs=pl.BlockSpec((blk, 1), lambda i: (i, 0)),
        out_shape=jax.ShapeDtypeStruct((b, 1), jnp.float32),
        compiler_params=pltpu.CompilerParams(
            dimension_semantics=("arbitrary",)),
    )(x_num, x_emb, w1n, w1e, b1, w2, b2, w3, b3)


def kernel(X_num, X_cat, tables, W1, b1, W2, b2, W3, b3):
    b = X_num.shape[0]
    flat_tables = tables.reshape(_NUM_FIELDS * _VOCAB, _EMB)
    offs = (jnp.arange(_NUM_FIELDS, dtype=jnp.int32) * _VOCAB)[None, :]
    flat_idx = (X_cat + offs).reshape(b * _NUM_FIELDS)

    x_emb = _sc_gather(flat_tables, flat_idx, b * _NUM_FIELDS, _EMB, 128)
    x_emb = x_emb.reshape(b, _NUM_FIELDS * _EMB)

    w1n = W1[:_NUM_NUM]
    w1e = W1[_NUM_NUM:]
    out = _tc_mlp(X_num, x_emb, w1n, w1e, b1.reshape(1, 32), W2,
                  b2.reshape(1, 16), W3, b3.reshape(1, 1), 512)
    return out
